# baseline (speedup 1.0000x reference)
import functools

import jax
import jax.numpy as jnp
from jax import lax
from jax.experimental import pallas as pl
from jax.experimental.pallas import tpu as pltpu
from jax.experimental.pallas import tpu_sc as plsc

_LANES = 16
_NUM_WORKERS = 32
_BLK_ROWS = 8


def _make_sc_call(shape, n_pts):
    b0, rows, cols = shape
    n_tab = n_pts + 2
    n_pad = n_tab + ((-n_tab) % _LANES)
    jmax = float(n_tab - 1)
    total_rows = b0 * rows
    rows_per_w = total_rows // _NUM_WORKERS
    n_chunks = rows_per_w // _BLK_ROWS
    vecs_per_row = cols // _LANES
    mesh = plsc.VectorSubcoreMesh(core_axis_name="c", subcore_axis_name="s")

    @functools.partial(
        pl.kernel,
        mesh=mesh,
        out_type=jax.ShapeDtypeStruct(shape, jnp.float32),
        compiler_params=pltpu.CompilerParams(needs_layout_passes=False),
        scratch_types=[
            pltpu.VMEM((n_pad,), jnp.float32),
            pltpu.VMEM((n_pad,), jnp.float32),
            pltpu.VMEM((n_pts,), jnp.float32),
            pltpu.VMEM((n_pts,), jnp.float32),
            pltpu.VMEM((3, _BLK_ROWS, cols), jnp.float32),
            pltpu.VMEM((2, _BLK_ROWS, cols), jnp.float32),
            pltpu.SemaphoreType.DMA,
            pltpu.SemaphoreType.DMA,
            pltpu.SemaphoreType.DMA,
            pltpu.SemaphoreType.DMA,
            pltpu.SemaphoreType.DMA,
        ],
    )
    def run(x_hbm, xv_hbm, yv_hbm, out_hbm, s_v, b_v, xv_v, yv_v, in_v, out_v,
            si0, si1, si2, so0, so1):
        cid = lax.axis_index("c")
        sid = lax.axis_index("s")
        wid = sid * 2 + cid
        base_row = wid * rows_per_w
        sin = (si0, si1, si2)
        sout = (so0, so1)

        def row0(k):
            return base_row + k * _BLK_ROWS

        pltpu.sync_copy(xv_hbm, xv_v)
        pltpu.sync_copy(yv_hbm, yv_v)
        in_dma = [None] * n_chunks
        out_dma = [None] * n_chunks
        for k0 in range(min(2, n_chunks)):
            in_dma[k0] = pltpu.async_copy(
                x_hbm.at[0, pl.ds(row0(k0), _BLK_ROWS), :],
                in_v.at[k0 % 3], sin[k0 % 3])
        iot = lax.iota(jnp.int32, _LANES)
        ones16 = jnp.full((_LANES,), 1, jnp.int32)
        x1v = plsc.load_gather(xv_v, [ones16])
        x2v = plsc.load_gather(xv_v, [ones16 + 1])
        hv = x2v - x1v
        for t in range(n_pad // _LANES):
            j = iot + (t * _LANES)
            ja = jnp.minimum(jnp.maximum(j, 1), n_pts)
            idx_a = jnp.minimum(ja, n_pts - 1)
            idx_b = ja - 1
            xa = plsc.load_gather(xv_v, [idx_a])
            ya = plsc.load_gather(yv_v, [idx_a])
            xb = plsc.load_gather(xv_v, [idx_b])
            yb = plsc.load_gather(yv_v, [idx_b])
            is_ramp = j == n_pts
            xa = jnp.where(is_ramp, xa + hv, xa)
            ya = jnp.where(is_ramp, xa, ya)
            s = (ya - yb) / (xa - xb)
            b = yb - s * xb
            ident = j >= n_pts + 1
            s = jnp.where(ident, 1.0, s)
            b = jnp.where(ident, 0.0, b)
            below = j == 0
            s = jnp.where(below, 0.0, s)
            b = jnp.where(below, 0.0, b)
            s_v[pl.ds(t * _LANES, _LANES)] = s
            b_v[pl.ds(t * _LANES, _LANES)] = b

        def compute(bi, bo):
            x1c = plsc.load_gather(xv_v, [ones16])
            x2c = plsc.load_gather(xv_v, [ones16 + 1])
            hc = x2c - x1c
            inv_h = 1.0 / hc
            cvec = 1.0 - (x1c - hc) * inv_h

            @plsc.parallel_loop(0, vecs_per_row, unroll=2)
            def vec_body(i):
                col = i * _LANES
                for r in range(_BLK_ROWS):
                    xv = in_v[bi, r, pl.ds(col, _LANES)]
                    t1 = xv * inv_h + cvec
                    t1 = jnp.minimum(jnp.maximum(t1, 0.0), jmax)
                    j = t1.astype(jnp.int32)
                    sv = plsc.load_gather(s_v, [j])
                    bv = plsc.load_gather(b_v, [j])
                    out_v[bo, r, pl.ds(col, _LANES)] = sv * xv + bv

        for k in range(n_chunks):
            bi = k % 3
            bo = k & 1
            if k + 2 < n_chunks:
                in_dma[k + 2] = pltpu.async_copy(
                    x_hbm.at[0, pl.ds(row0(k + 2), _BLK_ROWS), :],
                    in_v.at[(k + 2) % 3], sin[(k + 2) % 3])
            in_dma[k].wait()
            if k >= 2:
                out_dma[k - 2].wait()
            compute(bi, bo)
            out_dma[k] = pltpu.async_copy(
                out_v.at[bo], out_hbm.at[0, pl.ds(row0(k), _BLK_ROWS), :],
                sout[bo])
        for k in range(max(n_chunks - 2, 0), n_chunks):
            out_dma[k].wait()

    return run


def kernel(x, x_vals, y_vals):
    run = _make_sc_call(x.shape, x_vals.shape[0])
    return run(x, x_vals, y_vals)


# speedup vs baseline: 1.0399x; 1.0399x over previous
import functools

import jax
import jax.numpy as jnp
from jax import lax
from jax.experimental import pallas as pl
from jax.experimental.pallas import tpu as pltpu
from jax.experimental.pallas import tpu_sc as plsc

_LANES = 16
_NUM_WORKERS = 32
_BLK_ROWS = 8


def _make_sc_call(shape, n_pts):
    b0, rows, cols = shape
    n_tab = n_pts + 2
    n_pad = n_tab + ((-n_tab) % _LANES)
    jmax = float(n_tab - 1)
    total_rows = b0 * rows
    rows_per_w = total_rows // _NUM_WORKERS
    n_chunks = rows_per_w // _BLK_ROWS
    vecs_per_row = cols // _LANES
    mesh = plsc.VectorSubcoreMesh(core_axis_name="c", subcore_axis_name="s")

    @functools.partial(
        pl.kernel,
        mesh=mesh,
        out_type=jax.ShapeDtypeStruct(shape, jnp.float32),
        compiler_params=pltpu.CompilerParams(needs_layout_passes=False),
        scratch_types=[
            pltpu.VMEM((n_pad,), jnp.float32),
            pltpu.VMEM((n_pad,), jnp.float32),
            pltpu.VMEM((n_pts,), jnp.float32),
            pltpu.VMEM((n_pts,), jnp.float32),
            pltpu.VMEM((2, _BLK_ROWS, cols), jnp.float32),
            pltpu.VMEM((2, _BLK_ROWS, cols), jnp.float32),
            pltpu.SemaphoreType.DMA,
            pltpu.SemaphoreType.DMA,
            pltpu.SemaphoreType.DMA,
            pltpu.SemaphoreType.DMA,
        ],
    )
    def run(x_hbm, xv_hbm, yv_hbm, out_hbm, s_v, b_v, xv_v, yv_v, in_v, out_v,
            si0, si1, so0, so1):
        cid = lax.axis_index("c")
        sid = lax.axis_index("s")
        wid = sid * 2 + cid
        base_row = wid * rows_per_w
        sin = (si0, si1)
        sout = (so0, so1)

        def row0(k):
            return base_row + k * _BLK_ROWS

        pltpu.sync_copy(xv_hbm, xv_v)
        pltpu.sync_copy(yv_hbm, yv_v)
        in_dma = [None] * n_chunks
        out_dma = [None] * n_chunks
        in_dma[0] = pltpu.async_copy(
            x_hbm.at[0, pl.ds(row0(0), _BLK_ROWS), :], in_v.at[0], sin[0])
        iot = lax.iota(jnp.int32, _LANES)
        ones16 = jnp.full((_LANES,), 1, jnp.int32)
        x1v = plsc.load_gather(xv_v, [ones16])
        x2v = plsc.load_gather(xv_v, [ones16 + 1])
        hv = x2v - x1v
        for t in range(n_pad // _LANES):
            j = iot + (t * _LANES)
            ja = jnp.minimum(jnp.maximum(j, 1), n_pts)
            idx_a = jnp.minimum(ja, n_pts - 1)
            idx_b = ja - 1
            xa = plsc.load_gather(xv_v, [idx_a])
            ya = plsc.load_gather(yv_v, [idx_a])
            xb = plsc.load_gather(xv_v, [idx_b])
            yb = plsc.load_gather(yv_v, [idx_b])
            is_ramp = j == n_pts
            xa = jnp.where(is_ramp, xa + hv, xa)
            ya = jnp.where(is_ramp, xa, ya)
            s = (ya - yb) / (xa - xb)
            b = yb - s * xb
            ident = j >= n_pts + 1
            s = jnp.where(ident, 1.0, s)
            b = jnp.where(ident, 0.0, b)
            below = j == 0
            s = jnp.where(below, 0.0, s)
            b = jnp.where(below, 0.0, b)
            s_v[pl.ds(t * _LANES, _LANES)] = s
            b_v[pl.ds(t * _LANES, _LANES)] = b

        def compute(bi, bo):
            x1c = plsc.load_gather(xv_v, [ones16])
            x2c = plsc.load_gather(xv_v, [ones16 + 1])
            hc = x2c - x1c
            inv_h = 1.0 / hc
            cvec = 1.0 - (x1c - hc) * inv_h

            @plsc.parallel_loop(0, vecs_per_row, unroll=4)
            def vec_body(i):
                col = i * _LANES
                for r in range(_BLK_ROWS):
                    xv = in_v[bi, r, pl.ds(col, _LANES)]
                    t1 = xv * inv_h + cvec
                    t1 = jnp.minimum(jnp.maximum(t1, 0.0), jmax)
                    j = t1.astype(jnp.int32)
                    sv = plsc.load_gather(s_v, [j])
                    bv = plsc.load_gather(b_v, [j])
                    out_v[bo, r, pl.ds(col, _LANES)] = sv * xv + bv

        for k in range(n_chunks):
            bi = k & 1
            bo = k & 1
            if k + 1 < n_chunks:
                in_dma[k + 1] = pltpu.async_copy(
                    x_hbm.at[0, pl.ds(row0(k + 1), _BLK_ROWS), :],
                    in_v.at[1 - bi], sin[1 - bi])
            in_dma[k].wait()
            if k >= 2:
                out_dma[k - 2].wait()
            compute(bi, bo)
            out_dma[k] = pltpu.async_copy(
                out_v.at[bo], out_hbm.at[0, pl.ds(row0(k), _BLK_ROWS), :],
                sout[bo])
        for k in range(max(n_chunks - 2, 0), n_chunks):
            out_dma[k].wait()

    return run


def kernel(x, x_vals, y_vals):
    run = _make_sc_call(x.shape, x_vals.shape[0])
    return run(x, x_vals, y_vals)


# speedup vs baseline: 1.0959x; 1.0539x over previous
import functools

import jax
import jax.numpy as jnp
from jax import lax
from jax.experimental import pallas as pl
from jax.experimental.pallas import tpu as pltpu
from jax.experimental.pallas import tpu_sc as plsc

_LANES = 16
_NUM_WORKERS = 32
_BLK_ROWS = 8


def _make_sc_call(shape, n_pts):
    b0, rows, cols = shape
    n_tab = n_pts + 2
    n_pad = n_tab + ((-n_tab) % _LANES)
    jmax = float(n_tab - 1)
    total_rows = b0 * rows
    rows_per_w = total_rows // _NUM_WORKERS
    n_chunks = rows_per_w // _BLK_ROWS
    vecs_per_row = cols // _LANES
    mesh = plsc.VectorSubcoreMesh(core_axis_name="c", subcore_axis_name="s")

    @functools.partial(
        pl.kernel,
        mesh=mesh,
        out_type=jax.ShapeDtypeStruct(shape, jnp.float32),
        compiler_params=pltpu.CompilerParams(needs_layout_passes=False),
        scratch_types=[
            pltpu.VMEM((n_pad,), jnp.float32),
            pltpu.VMEM((n_pad,), jnp.float32),
            pltpu.VMEM((n_pts,), jnp.float32),
            pltpu.VMEM((n_pts,), jnp.float32),
            pltpu.VMEM((2, _BLK_ROWS, cols), jnp.float32),
            pltpu.VMEM((2, _BLK_ROWS, cols), jnp.float32),
            pltpu.SemaphoreType.DMA,
            pltpu.SemaphoreType.DMA,
            pltpu.SemaphoreType.DMA,
            pltpu.SemaphoreType.DMA,
        ],
    )
    def run(x_hbm, xv_hbm, yv_hbm, out_hbm, s_v, b_v, xv_v, yv_v, in_v, out_v,
            si0, si1, so0, so1):
        cid = lax.axis_index("c")
        sid = lax.axis_index("s")
        wid = sid * 2 + cid
        base_row = wid * rows_per_w
        sin = (si0, si1)
        sout = (so0, so1)

        def row0(k):
            return base_row + k * _BLK_ROWS

        pltpu.sync_copy(xv_hbm, xv_v)
        pltpu.sync_copy(yv_hbm, yv_v)
        in_dma = [None] * n_chunks
        out_dma = [None] * n_chunks
        in_dma[0] = pltpu.async_copy(
            x_hbm.at[0, pl.ds(row0(0), _BLK_ROWS), :], in_v.at[0], sin[0])
        iot = lax.iota(jnp.int32, _LANES)
        ones16 = jnp.full((_LANES,), 1, jnp.int32)
        x1v = plsc.load_gather(xv_v, [ones16])
        x2v = plsc.load_gather(xv_v, [ones16 + 1])
        hv = x2v - x1v
        for t in range(n_pad // _LANES):
            j = iot + (t * _LANES)
            ja = jnp.minimum(jnp.maximum(j, 1), n_pts)
            idx_a = jnp.minimum(ja, n_pts - 1)
            idx_b = ja - 1
            xa = plsc.load_gather(xv_v, [idx_a])
            ya = plsc.load_gather(yv_v, [idx_a])
            xb = plsc.load_gather(xv_v, [idx_b])
            yb = plsc.load_gather(yv_v, [idx_b])
            is_ramp = j == n_pts
            xa = jnp.where(is_ramp, xa + hv, xa)
            ya = jnp.where(is_ramp, xa, ya)
            s = (ya - yb) / (xa - xb)
            b = yb - s * xb
            ident = j >= n_pts + 1
            s = jnp.where(ident, 1.0, s)
            b = jnp.where(ident, 0.0, b)
            below = j == 0
            s = jnp.where(below, 0.0, s)
            b = jnp.where(below, 0.0, b)
            s_v[pl.ds(t * _LANES, _LANES)] = s
            b_v[pl.ds(t * _LANES, _LANES)] = b

        def compute(bi, bo):
            x1c = plsc.load_gather(xv_v, [ones16])
            x2c = plsc.load_gather(xv_v, [ones16 + 1])
            hc = x2c - x1c
            inv_h = 1.0 / hc
            cvec = 1.0 - (x1c - hc) * inv_h

            @plsc.parallel_loop(0, vecs_per_row, unroll=2)
            def vec_body(i):
                col = i * _LANES
                for r in range(_BLK_ROWS):
                    xv = in_v[bi, r, pl.ds(col, _LANES)]
                    t1 = xv * inv_h + cvec
                    t1 = jnp.minimum(jnp.maximum(t1, 0.0), jmax)
                    j = t1.astype(jnp.int32)
                    sv = plsc.load_gather(s_v, [j])
                    bv = plsc.load_gather(b_v, [j])
                    out_v[bo, r, pl.ds(col, _LANES)] = sv * xv + bv

        for k in range(n_chunks):
            bi = k & 1
            bo = k & 1
            if k + 1 < n_chunks:
                in_dma[k + 1] = pltpu.async_copy(
                    x_hbm.at[0, pl.ds(row0(k + 1), _BLK_ROWS), :],
                    in_v.at[1 - bi], sin[1 - bi])
            in_dma[k].wait()
            if k >= 2:
                out_dma[k - 2].wait()
            compute(bi, bo)
            out_dma[k] = pltpu.async_copy(
                out_v.at[bo], out_hbm.at[0, pl.ds(row0(k), _BLK_ROWS), :],
                sout[bo])
        for k in range(max(n_chunks - 2, 0), n_chunks):
            out_dma[k].wait()

    return run


def kernel(x, x_vals, y_vals):
    run = _make_sc_call(x.shape, x_vals.shape[0])
    return run(x, x_vals, y_vals)
